# baseline (speedup 1.0000x reference)
import functools

import jax
import jax.numpy as jnp
from jax import lax
from jax.experimental import pallas as pl
from jax.experimental.pallas import tpu as pltpu
from jax.experimental.pallas import tpu_sc as plsc

_NC = 2
_NS = 16
_NW = _NC * _NS
_CH = 640
_NBUF = 2


@functools.cache
def _make(R, H, D):
    rpw = R // _NW
    rch = 16
    nch = rpw // rch
    ch = rch * H
    hp = 8 * ((H + 7) // 8)
    mesh = plsc.VectorSubcoreMesh(core_axis_name="c", subcore_axis_name="s")

    @functools.partial(
        pl.kernel,
        out_type=jax.ShapeDtypeStruct((R, hp, 2 * D), jnp.float32),
        mesh=mesh,
        scratch_types=[
            pltpu.VMEM((nch, ch), jnp.int32),
            pltpu.VMEM((_NBUF, ch, D), jnp.float32),
            [pltpu.SemaphoreType.DMA] * _NBUF,
            [pltpu.SemaphoreType.DMA] * _NBUF,
        ],
        compiler_params=pltpu.CompilerParams(use_tc_tiling_on_sc=False),
    )
    def emb(idx_hbm, table_hbm, out_hbm, idx_v, rows_v, sems_g, sems_s):
        wid = lax.axis_index("s") * _NC + lax.axis_index("c")
        pltpu.sync_copy(idx_hbm.at[wid], idx_v)
        rbase = wid * rpw

        def gather(j):
            return pltpu.async_copy(
                table_hbm.at[idx_v.at[j]], rows_v.at[j % _NBUF],
                sems_g[j % _NBUF])

        def scatter(j):
            hs = []
            for k in range(rch):
                hs.append(pltpu.async_copy(
                    rows_v.at[j % _NBUF].at[pl.ds(k * H, H)],
                    out_hbm.at[rbase + j * rch + k, pl.ds(0, H), pl.ds(0, D)],
                    sems_s[j % _NBUF]))
            return hs

        h_g = [None] * nch
        h_s = [None] * nch
        h_g[0] = gather(0)
        for j in range(nch):
            if j + 1 < nch:
                if j - (_NBUF - 1) >= 0:
                    for h in h_s[j - (_NBUF - 1)]:
                        h.wait()
                h_g[j + 1] = gather(j + 1)
            h_g[j].wait()
            h_s[j] = scatter(j)
        for j in range(max(0, nch - _NBUF + 1), nch):
            for h in h_s[j]:
                h.wait()

    return emb


@jax.jit
def kernel(input_ids, weight):
    R, H = input_ids.shape
    V, D = weight.shape
    rpw = R // _NW
    wpad2 = jnp.pad(weight, ((0, 0), (0, D))).reshape(2 * V, D)
    idx3 = (input_ids.astype(jnp.int32) * 2).reshape(_NW, rpw // 16, 16 * H)
    out = _make(R, H, D)(idx3, wpad2)
    return out[:, :H, :D]


# speedup vs baseline: 1.0023x; 1.0023x over previous
import functools

import jax
import jax.numpy as jnp
from jax import lax
from jax.experimental import pallas as pl
from jax.experimental.pallas import tpu as pltpu
from jax.experimental.pallas import tpu_sc as plsc

_NC = 2
_NS = 16
_NW = _NC * _NS
_NBUF = 3


@functools.cache
def _make(R, H, D):
    rpw = R // _NW
    rch = 8
    nch = rpw // rch
    ch = rch * H
    hp = 8 * ((H + 7) // 8)
    mesh = plsc.VectorSubcoreMesh(core_axis_name="c", subcore_axis_name="s")

    @functools.partial(
        pl.kernel,
        out_type=jax.ShapeDtypeStruct((R, hp, 2 * D), jnp.float32),
        mesh=mesh,
        scratch_types=[
            pltpu.VMEM((nch, ch), jnp.int32),
            pltpu.VMEM((_NBUF, ch, D), jnp.float32),
            [pltpu.SemaphoreType.DMA] * _NBUF,
            [pltpu.SemaphoreType.DMA] * _NBUF,
        ],
        compiler_params=pltpu.CompilerParams(use_tc_tiling_on_sc=False),
    )
    def emb(idx_hbm, table_hbm, out_hbm, idx_v, rows_v, sems_g, sems_s):
        wid = lax.axis_index("s") * _NC + lax.axis_index("c")
        pltpu.sync_copy(idx_hbm.at[wid], idx_v)
        rbase = wid * rpw

        def gather(j):
            return pltpu.async_copy(
                table_hbm.at[idx_v.at[j]], rows_v.at[j % _NBUF],
                sems_g[j % _NBUF])

        def scatter(j):
            hs = []
            for k in range(rch):
                hs.append(pltpu.async_copy(
                    rows_v.at[j % _NBUF].at[pl.ds(k * H, H)],
                    out_hbm.at[rbase + j * rch + k, pl.ds(0, H), pl.ds(0, D)],
                    sems_s[j % _NBUF]))
            return hs

        h_g = [None] * nch
        h_s = [None] * nch
        h_g[0] = gather(0)
        for j in range(nch):
            if j + 1 < nch:
                if j - (_NBUF - 1) >= 0:
                    for h in h_s[j - (_NBUF - 1)]:
                        h.wait()
                h_g[j + 1] = gather(j + 1)
            h_g[j].wait()
            h_s[j] = scatter(j)
        for j in range(max(0, nch - _NBUF + 1), nch):
            for h in h_s[j]:
                h.wait()

    return emb


@jax.jit
def kernel(input_ids, weight):
    R, H = input_ids.shape
    V, D = weight.shape
    rpw = R // _NW
    wpad2 = jnp.pad(weight, ((0, 0), (0, D))).reshape(2 * V, D)
    idx3 = (input_ids.astype(jnp.int32) * 2).reshape(_NW, rpw // 8, 8 * H)
    out = _make(R, H, D)(idx3, wpad2)
    return out[:, :H, :D]
